# baseline (speedup 1.0000x reference)
import functools

import jax
import jax.numpy as jnp
from jax import lax
from jax.experimental import pallas as pl
from jax.experimental.pallas import tpu as pltpu
from jax.experimental.pallas import tpu_sc as plsc

N_NODES = 10000
N_EDGES = 320000
D = 128

NC = 2
NS = 16
NW = NC * NS
E_PER_W = N_EDGES // NW
CH = 64
NCHUNK = 160
E_PAD_W = NCHUNK * CH
N_PAD = 10112
ROWS_PER_TILE = N_PAD // NS

_MESH = plsc.VectorSubcoreMesh(
    core_axis_name="c", subcore_axis_name="s", num_cores=NC, num_subcores=NS
)


@functools.partial(
    pl.kernel,
    out_type=jax.ShapeDtypeStruct((NC, N_PAD, D), jnp.float32),
    mesh=_MESH,
    scratch_types=[
        pltpu.VMEM((NCHUNK, CH), jnp.int32),
        pltpu.VMEM((NCHUNK, CH), jnp.int32),
        pltpu.VMEM((1, CH, D), jnp.float32),
        pltpu.VMEM_SHARED((N_PAD, D), jnp.float32),
        pltpu.SemaphoreType.DMA((2,)),
    ],
)
def _sc_segment_sum(src_hbm, dst_hbm, x_hbm, zeros_hbm, part_hbm,
                    src_v, dst_v, rows_v, h_sh, ssem):
    c = lax.axis_index("c")
    s = lax.axis_index("s")
    wid = c * NS + s

    pltpu.sync_copy(src_hbm.at[wid], src_v)
    pltpu.sync_copy(dst_hbm.at[wid], dst_v)

    row0 = s * ROWS_PER_TILE
    pltpu.sync_copy(zeros_hbm.at[pl.ds(row0, ROWS_PER_TILE)],
                    h_sh.at[pl.ds(row0, ROWS_PER_TILE)])
    plsc.subcore_barrier()

    def gather(j, b):
        pltpu.sync_copy(x_hbm.at[src_v.at[j]], rows_v.at[b])

    def scatter_start(j, b):
        pltpu.async_copy(rows_v.at[b], h_sh.at[dst_v.at[j]], ssem.at[b],
                         add=True)

    def scatter_wait(j, b):
        pltpu.make_async_copy(rows_v.at[b], h_sh.at[dst_v.at[j]],
                              ssem.at[b]).wait()

    def step(j, carry):
        gather(j, 0)
        pltpu.sync_copy(rows_v.at[0], h_sh.at[dst_v.at[j]], add=True)
        return carry

    lax.fori_loop(0, NCHUNK, step, 0)

    plsc.subcore_barrier()
    pltpu.sync_copy(h_sh.at[pl.ds(row0, ROWS_PER_TILE)],
                    part_hbm.at[c, pl.ds(row0, ROWS_PER_TILE)])


def _tc_linear_body(p0_ref, p1_ref, wt_ref, b_ref, o_ref):
    h = p0_ref[...] + p1_ref[...]
    o_ref[...] = (
        jnp.dot(h, wt_ref[...], preferred_element_type=jnp.float32) + b_ref[...]
    )


def _tc_linear(p0, p1, w_t, b2d):
    blk = N_PAD // 16
    grid = N_PAD // blk
    return pl.pallas_call(
        _tc_linear_body,
        grid=(grid,),
        in_specs=[
            pl.BlockSpec((blk, D), lambda i: (i, 0)),
            pl.BlockSpec((blk, D), lambda i: (i, 0)),
            pl.BlockSpec((D, D), lambda i: (0, 0)),
            pl.BlockSpec((1, D), lambda i: (0, 0)),
        ],
        out_specs=pl.BlockSpec((blk, D), lambda i: (i, 0)),
        out_shape=jax.ShapeDtypeStruct((N_PAD, D), jnp.float32),
    )(p0, p1, w_t, b2d)


def kernel(inputs, edge_index, W, b):
    pad = ((0, 0), (0, E_PAD_W - E_PER_W))
    src = jnp.pad(edge_index[0].reshape(NW, E_PER_W), pad,
                  constant_values=0).reshape(NW, NCHUNK, CH)
    dst = jnp.pad(edge_index[1].reshape(NW, E_PER_W), pad,
                  constant_values=N_NODES).reshape(NW, NCHUNK, CH)
    zeros = jnp.zeros((N_PAD, D), jnp.float32)
    partials = _sc_segment_sum(src, dst, inputs, zeros)
    out = _tc_linear(partials[0], partials[1], W.T, b.reshape(1, D))
    return out[:N_NODES]


# speedup vs baseline: 7.4402x; 7.4402x over previous
import functools

import jax
import jax.numpy as jnp
from jax import lax
from jax.experimental import pallas as pl
from jax.experimental.pallas import tpu as pltpu
from jax.experimental.pallas import tpu_sc as plsc

N_NODES = 10000
N_EDGES = 320000
D = 128

NC = 2
NS = 16
NW = NC * NS
E_PER_W = N_EDGES // NW
CH = 80
NCHUNK = 125
E_PAD_W = NCHUNK * CH
N_PAD = 10240
ROWS_PER_TILE = N_PAD // NS

_MESH = plsc.VectorSubcoreMesh(
    core_axis_name="c", subcore_axis_name="s", num_cores=NC, num_subcores=NS
)


@functools.partial(
    pl.kernel,
    out_type=jax.ShapeDtypeStruct((NC, N_PAD, D), jnp.float32),
    mesh=_MESH,
    scratch_types=[
        pltpu.VMEM((E_PAD_W,), jnp.int32),
        pltpu.VMEM((NCHUNK, CH), jnp.int32),
        pltpu.VMEM((2, CH, D), jnp.float32),
        pltpu.VMEM_SHARED((N_PAD, D), jnp.float32),
        pltpu.SemaphoreType.DMA((2,)),
    ],
)
def _sc_segment_sum(src_hbm, dst_hbm, x_hbm, zeros_hbm, part_hbm,
                    src_v, dst_v, rows_v, h_sh, ssem):
    c = lax.axis_index("c")
    s = lax.axis_index("s")
    wid = c * NS + s

    pltpu.sync_copy(src_hbm.at[wid], src_v)
    pltpu.sync_copy(dst_hbm.at[wid], dst_v)

    row0 = s * ROWS_PER_TILE
    pltpu.sync_copy(zeros_hbm.at[pl.ds(row0, ROWS_PER_TILE)],
                    h_sh.at[pl.ds(row0, ROWS_PER_TILE)])
    plsc.subcore_barrier()

    def gather(j, b):
        pltpu.sync_copy(x_hbm.at[src_v.at[pl.ds(j * CH, CH)]], rows_v.at[b])

    def scatter_start(j, b):
        pltpu.async_copy(rows_v.at[b], h_sh.at[dst_v.at[j]], ssem.at[b],
                         add=True)

    def scatter_wait(j, b):
        pltpu.make_async_copy(rows_v.at[b], h_sh.at[dst_v.at[j]],
                              ssem.at[b]).wait()

    @functools.partial(plsc.parallel_loop, 0, NCHUNK, unroll=2)
    def _loop(j):
        b = lax.rem(j, 2)
        gather(j, b)
        pltpu.sync_copy(rows_v.at[b], h_sh.at[dst_v.at[j]], add=True)

    plsc.subcore_barrier()
    pltpu.sync_copy(h_sh.at[pl.ds(row0, ROWS_PER_TILE)],
                    part_hbm.at[c, pl.ds(row0, ROWS_PER_TILE)])


def _tc_linear_body(p0_ref, p1_ref, wt_ref, b_ref, o_ref):
    h = p0_ref[...] + p1_ref[...]
    o_ref[...] = (
        jnp.dot(h, wt_ref[...], preferred_element_type=jnp.float32) + b_ref[...]
    )


def _tc_linear(p0, p1, w_t, b2d):
    blk = N_PAD // 16
    grid = N_PAD // blk
    return pl.pallas_call(
        _tc_linear_body,
        grid=(grid,),
        in_specs=[
            pl.BlockSpec((blk, D), lambda i: (i, 0)),
            pl.BlockSpec((blk, D), lambda i: (i, 0)),
            pl.BlockSpec((D, D), lambda i: (0, 0)),
            pl.BlockSpec((1, D), lambda i: (0, 0)),
        ],
        out_specs=pl.BlockSpec((blk, D), lambda i: (i, 0)),
        out_shape=jax.ShapeDtypeStruct((N_PAD, D), jnp.float32),
    )(p0, p1, w_t, b2d)


def kernel(inputs, edge_index, W, b):
    pad = ((0, 0), (0, E_PAD_W - E_PER_W))
    src = jnp.pad(edge_index[0].reshape(NW, E_PER_W), pad, constant_values=0)
    dst = jnp.pad(edge_index[1].reshape(NW, E_PER_W), pad,
                  constant_values=N_NODES).reshape(NW, NCHUNK, CH)
    zeros = jnp.zeros((N_PAD, D), jnp.float32)
    partials = _sc_segment_sum(src, dst, inputs, zeros)
    out = _tc_linear(partials[0], partials[1], W.T, b.reshape(1, D))
    return out[:N_NODES]
